# baseline (speedup 1.0000x reference)
import functools

import jax
import jax.numpy as jnp
from jax import lax
from jax.experimental import pallas as pl
from jax.experimental.pallas import tpu as pltpu
from jax.experimental.pallas import tpu_sc as plsc

NC = 2
NS = 16
L = 16
K = 88
BR = 2000


def _mesh():
    return plsc.VectorSubcoreMesh(core_axis_name="c", subcore_axis_name="s")


@functools.cache
def _agg_kernel(tab_rows, et, n_acc, f):
    nb = et // K
    R = 4
    QI = 8
    assert et % (QI * K) == 0 and n_acc % NS == 0
    rpt = n_acc // NS
    chunks = [(t * K, K) for t in range(rpt // K)]
    if rpt % K:
        chunks.append((rpt // K * K, rpt % K))

    @functools.partial(
        pl.kernel,
        out_type=jax.ShapeDtypeStruct((NC * n_acc, f), jnp.float32),
        mesh=_mesh(),
        scratch_types=[
            [pltpu.VMEM((K,), jnp.int32) for _ in range(QI)],
            [pltpu.VMEM((K,), jnp.int32) for _ in range(QI)],
            [pltpu.VMEM((K, f), jnp.float32) for _ in range(R)],
            pltpu.VMEM_SHARED((n_acc, f), jnp.float32),
            [pltpu.SemaphoreType.DMA for _ in range(QI)],
            [pltpu.SemaphoreType.DMA for _ in range(QI)],
            [pltpu.SemaphoreType.DMA for _ in range(R)],
            [pltpu.SemaphoreType.DMA for _ in range(R)],
        ],
    )
    def agg(table, src, dst, zrows, out,
            srcv, dstv, rows, acc, isem, jsem, gsem, ssem):
        c = lax.axis_index("c")
        s = lax.axis_index("s")

        pltpu.sync_copy(zrows, rows[0])
        for off, sz in chunks:
            pltpu.async_copy(
                rows[0].at[pl.ds(0, sz)],
                acc.at[pl.ds(s * rpt + off, sz)], gsem[0])
        for off, sz in chunks:
            pltpu.make_async_copy(
                rows[0].at[pl.ds(0, sz)],
                acc.at[pl.ds(s * rpt, sz)], gsem[0]).wait()
        plsc.subcore_barrier()

        ebase = (c * NS + s) * et

        def fire_idx(b, q):
            o = pl.multiple_of(ebase + b * K, K)
            pltpu.async_copy(src.at[pl.ds(o, K)], srcv[q], isem[q])
            pltpu.async_copy(dst.at[pl.ds(o, K)], dstv[q], jsem[q])

        def wait(sem, src_ref, dst_ref):
            pltpu.make_async_copy(src_ref, dst_ref, sem).wait()

        def fire_gather(b, q, j):
            o = pl.multiple_of(ebase + b * K, K)
            wait(isem[q], src.at[pl.ds(o, K)], srcv[q])
            pltpu.async_copy(table.at[srcv[q]], rows[j], gsem[j])

        for q in range(QI // 2):
            fire_idx(q, q)
        fire_gather(0, 0, 0)
        fire_gather(1, 1, 1)

        @pl.loop(0, nb, step=QI)
        def _edges(i):
            for u in range(QI):
                b = i + u
                j = u % R
                q = u
                j2 = (u + 2) % R
                q2 = (u + 2) % QI
                qr = (u + 4) % QI
                ob = pl.multiple_of(ebase + b * K, K)

                wait(gsem[j], table.at[srcv[q]], rows[j])
                wait(jsem[q], dst.at[pl.ds(ob, K)], dstv[q])
                pltpu.async_copy(rows[j], acc.at[dstv[q]], ssem[j], add=True)

                @pl.when(b + 2 < nb)
                def _next_gather():
                    @pl.when(b >= 2)
                    def _slot_free():
                        wait(ssem[j2], rows[j2], acc.at[dstv[q2]])
                    fire_gather(b + 2, q2, j2)

                @pl.when(b + 4 < nb)
                def _refill():
                    fire_idx(b + 4, qr)

        for j in range(R):
            wait(ssem[j], rows[j], acc.at[dstv[QI // 2 + j]])
        plsc.subcore_barrier()

        for t, (off, sz) in enumerate(chunks):
            j = t % 2
            if t >= 2:
                po, psz = chunks[t - 2]
                wait(gsem[j], rows[j].at[pl.ds(0, psz)],
                     out.at[pl.ds(c * n_acc, psz)])
            r0 = s * rpt + off
            pltpu.sync_copy(acc.at[pl.ds(r0, sz)], rows[j].at[pl.ds(0, sz)])
            pltpu.async_copy(rows[j].at[pl.ds(0, sz)],
                             out.at[pl.ds(c * n_acc + r0, sz)], gsem[j])
        for t in range(max(0, len(chunks) - 2), len(chunks)):
            off, sz = chunks[t]
            wait(gsem[t % 2], rows[t % 2].at[pl.ds(0, sz)],
                 out.at[pl.ds(c * n_acc, sz)])

    return agg


KD = 128


@functools.cache
def _deg_kernel(et, nh):
    nb = et // KD
    seg = nh // NS
    assert nh % (NS * L) == 0

    @functools.partial(
        pl.kernel,
        out_type=jax.ShapeDtypeStruct((NC * nh,), jnp.float32),
        mesh=_mesh(),
        compiler_params=pltpu.CompilerParams(needs_layout_passes=False),
        scratch_types=[
            pltpu.VMEM((nh,), jnp.float32),
            [pltpu.VMEM((KD,), jnp.int32) for _ in range(2)],
            pltpu.VMEM((seg,), jnp.float32),
            pltpu.VMEM((seg,), jnp.float32),
            pltpu.VMEM_SHARED((NS * nh,), jnp.float32),
            [pltpu.SemaphoreType.DMA for _ in range(2)],
        ],
    )
    def degk(dst, out, hist, dstv, tmp, accv, spm, dsem):
        c = lax.axis_index("c")
        s = lax.axis_index("s")

        @pl.loop(0, nh // L)
        def _zero(i):
            hist[pl.ds(pl.multiple_of(i * L, L), L)] = jnp.zeros((L,), jnp.float32)

        ebase = (c * NS + s) * et

        def fire(i, j):
            off = pl.multiple_of(ebase + i * KD, KD)
            pltpu.async_copy(dst.at[pl.ds(off, KD)], dstv[j], dsem[j])

        fire(0, 0)

        @pl.loop(0, nb, step=2)
        def _count(i):
            for u in range(2):
                b = i + u
                j = u
                o = pl.multiple_of(ebase + b * KD, KD)
                pltpu.make_async_copy(dst.at[pl.ds(o, KD)], dstv[j],
                                      dsem[j]).wait()

                @pl.when(b + 1 < nb)
                def _pf():
                    fire(b + 1, 1 - j)
                for k in range(KD // L):
                    idx = dstv[j][pl.ds(k * L, L)]
                    plsc.addupdate_scatter(hist, [idx],
                                           jnp.ones((L,), jnp.float32))

        pltpu.sync_copy(hist, spm.at[pl.ds(s * nh, nh)])
        plsc.subcore_barrier()

        pltpu.sync_copy(spm.at[pl.ds(s * seg, seg)], accv)
        for j in range(1, NS):
            pltpu.sync_copy(spm.at[pl.ds(j * nh + s * seg, seg)], tmp)

            @pl.loop(0, seg // L)
            def _red(m):
                o = pl.multiple_of(m * L, L)
                accv[pl.ds(o, L)] = accv[pl.ds(o, L)] + tmp[pl.ds(o, L)]

        pltpu.sync_copy(accv, out.at[pl.ds(c * nh + s * seg, seg)])

    return degk


def _tc_params():
    return pltpu.CompilerParams(vmem_limit_bytes=60 * 1024 * 1024)


def _dot(a, b):
    return lax.dot_general(a, b, (((1,), (0,)), ((), ())),
                           precision=lax.Precision.HIGHEST)


def kernel(x, edge_index, W1, b1, g1, bt1, W2, b2, g2, bt2, W3, b3):
    n, fin = x.shape
    e = edge_index.shape[1]
    h = W1.shape[1]
    fo = W3.shape[1]
    n_acc = -(-n // (NS * 8)) * (NS * 8)
    nh = -(-n // (NS * L)) * (NS * L)
    src = edge_index[0].astype(jnp.int32)
    dst = edge_index[1].astype(jnp.int32)
    zrows = jnp.zeros((K, fin), jnp.float32)
    nblk = n // BR
    assert n % BR == 0

    def chunked(vals, T, et, is_dst):
        k = jnp.arange(et, dtype=jnp.int32)[None, :]
        padv = n + (k % (n_acc - n)) if is_dst else k % K
        if e % T == 0:
            body = vals.reshape(T, e // T)
            padcols = jnp.broadcast_to(padv[:, e // T:], (T, et - e // T))
            return jnp.concatenate([body, padcols], axis=1).reshape(-1)
        base, rem = divmod(e, T)
        t = jnp.arange(T, dtype=jnp.int32)[:, None]
        cnt = base + (t < rem).astype(jnp.int32)
        ridx = t * base + jnp.minimum(t, rem) + k
        return jnp.where(k < cnt, vals[jnp.clip(ridx, 0, e - 1)],
                         padv).reshape(-1)

    et_deg = -(-e // (NC * NS * 2 * KD)) * (2 * KD)
    dst_deg = chunked(dst, NC * NS, et_deg, True)

    et_es = -(-e // (NC * NS * 8 * K)) * (8 * K)
    src_es = chunked(src, NC * NS, et_es, False)
    dst_es = chunked(dst, NC * NS, et_es, True)

    et_fs = -(-e // (NS * 8 * K)) * (8 * K)
    srcp = chunked(src, NS, et_fs, False)
    dstp = chunked(dst, NS, et_fs, True)
    src_fs = jnp.concatenate([srcp, srcp + n])
    dst_fs = jnp.concatenate([dstp, dstp])

    col = lambda w: pl.BlockSpec((1, w), lambda i: (0, 0))
    rows = lambda w: pl.BlockSpec((BR, w), lambda i: (i, 0))
    pair = pl.BlockSpec((NC, BR, fin), lambda i: (0, i, 0))

    degp = _deg_kernel(et_deg, nh)(dst_deg).reshape(NC, nh, 1)

    def t1(degp_ref, x_ref, dinv_ref, y0_ref):
        deg = degp_ref[0] + degp_ref[1] + 1.0
        dinv = jnp.where(deg > 0.0, lax.rsqrt(deg), 0.0)
        dinv_ref[...] = dinv
        y0_ref[...] = dinv * x_ref[...]

    dinv, y0 = pl.pallas_call(
        t1,
        grid=(nblk,),
        in_specs=[pl.BlockSpec((NC, BR, 1), lambda i: (0, i, 0)), rows(fin)],
        out_specs=[rows(1), rows(fin)],
        out_shape=[jax.ShapeDtypeStruct((n, 1), jnp.float32),
                   jax.ShapeDtypeStruct((n, fin), jnp.float32)],
        compiler_params=_tc_params(),
    )(degp, x)

    s0 = _agg_kernel(n, et_es, n_acc, fin)(y0, src_es, dst_es, zrows)
    s0 = s0.reshape(NC, n_acc, fin)

    def accum_stats(z, s_ref, q_ref):
        @pl.when(pl.program_id(0) == 0)
        def _init():
            s_ref[...] = jnp.zeros_like(s_ref)
            q_ref[...] = jnp.zeros_like(q_ref)
        s_ref[...] += jnp.sum(z, axis=0, keepdims=True)
        q_ref[...] += jnp.sum(z * z, axis=0, keepdims=True)

    def bn_relu(z, sm, sq, g, bt):
        m = sm * (1.0 / n)
        v = sq * (1.0 / n) - m * m
        return jnp.maximum((z - m) / jnp.sqrt(v + 1e-5) * g + bt, 0.0)

    def t2a(s0_ref, y0_ref, dinv_ref, w1_ref, b1_ref, z1_ref, sm_ref, sq_ref):
        agg0 = dinv_ref[...] * (s0_ref[0] + s0_ref[1] + y0_ref[...])
        z1 = _dot(agg0, w1_ref[...]) + b1_ref[...]
        z1_ref[...] = z1
        accum_stats(z1, sm_ref, sq_ref)

    z1, m1, q1 = pl.pallas_call(
        t2a,
        grid=(nblk,),
        in_specs=[pair, rows(fin), rows(1),
                  pl.BlockSpec((fin, h), lambda i: (0, 0)), col(h)],
        out_specs=[rows(h), col(h), col(h)],
        out_shape=[jax.ShapeDtypeStruct((n, h), jnp.float32),
                   jax.ShapeDtypeStruct((1, h), jnp.float32),
                   jax.ShapeDtypeStruct((1, h), jnp.float32)],
        compiler_params=_tc_params(),
    )(s0, y0, dinv, W1, b1.reshape(1, h))

    def t2b(z_ref, s_ref, q_ref, g_ref, bt_ref, dinv_ref, w_ref, y2_ref):
        h1 = bn_relu(z_ref[...], s_ref[...], q_ref[...], g_ref[...],
                     bt_ref[...])
        y2 = dinv_ref[...] * _dot(h1, w_ref[...])
        y2_ref[0] = y2[:, :fin]
        y2_ref[1] = y2[:, fin:]

    y2pair = pl.pallas_call(
        t2b,
        grid=(nblk,),
        in_specs=[rows(h), col(h), col(h), col(h), col(h), rows(1),
                  pl.BlockSpec((h, h), lambda i: (0, 0))],
        out_specs=pair,
        out_shape=jax.ShapeDtypeStruct((NC, n, fin), jnp.float32),
        compiler_params=_tc_params(),
    )(z1, m1, q1, g1.reshape(1, h), bt1.reshape(1, h), dinv, W2)

    s2 = _agg_kernel(NC * n, et_fs, n_acc, fin)(
        y2pair.reshape(NC * n, fin), src_fs, dst_fs, zrows)
    s2 = s2.reshape(NC, n_acc, fin)

    def t3a(s2_ref, y2_ref, dinv_ref, b2_ref, z2_ref, sm_ref, sq_ref):
        ssum = jnp.concatenate(
            [s2_ref[0] + y2_ref[0], s2_ref[1] + y2_ref[1]], axis=1)
        z2 = dinv_ref[...] * ssum + b2_ref[...]
        z2_ref[...] = z2
        accum_stats(z2, sm_ref, sq_ref)

    z2, m2, q2 = pl.pallas_call(
        t3a,
        grid=(nblk,),
        in_specs=[pair, pair, rows(1), col(h)],
        out_specs=[rows(h), col(h), col(h)],
        out_shape=[jax.ShapeDtypeStruct((n, h), jnp.float32),
                   jax.ShapeDtypeStruct((1, h), jnp.float32),
                   jax.ShapeDtypeStruct((1, h), jnp.float32)],
        compiler_params=_tc_params(),
    )(s2, y2pair, dinv, b2.reshape(1, h))

    def t3b(z_ref, s_ref, q_ref, g_ref, bt_ref, dinv_ref, w_ref, y3_ref):
        h2 = bn_relu(z_ref[...], s_ref[...], q_ref[...], g_ref[...],
                     bt_ref[...])
        y3_ref[...] = dinv_ref[...] * _dot(h2, w_ref[...])

    y3 = pl.pallas_call(
        t3b,
        grid=(nblk,),
        in_specs=[rows(h), col(h), col(h), col(h), col(h), rows(1),
                  pl.BlockSpec((h, fo), lambda i: (0, 0))],
        out_specs=rows(fo),
        out_shape=jax.ShapeDtypeStruct((n, fo), jnp.float32),
        compiler_params=_tc_params(),
    )(z2, m2, q2, g2.reshape(1, h), bt2.reshape(1, h), dinv, W3)

    s3 = _agg_kernel(n, et_es, n_acc, fo)(y3, src_es, dst_es, zrows)
    s3 = s3.reshape(NC, n_acc, fo)

    def t4(s3_ref, y3_ref, dinv_ref, b3_ref, o_ref):
        o_ref[...] = dinv_ref[...] * (s3_ref[0] + s3_ref[1] + y3_ref[...]) \
            + b3_ref[...]

    out = pl.pallas_call(
        t4,
        grid=(nblk,),
        in_specs=[pl.BlockSpec((NC, BR, fo), lambda i: (0, i, 0)), rows(fo),
                  rows(1), col(fo)],
        out_specs=rows(fo),
        out_shape=jax.ShapeDtypeStruct((n, fo), jnp.float32),
        compiler_params=_tc_params(),
    )(s3, y3, dinv, b3.reshape(1, fo))
    return out


# speedup vs baseline: 1.0140x; 1.0140x over previous
import functools

import jax
import jax.numpy as jnp
from jax import lax
from jax.experimental import pallas as pl
from jax.experimental.pallas import tpu as pltpu
from jax.experimental.pallas import tpu_sc as plsc

NC = 2
NS = 16
L = 16
K = 80
BR = 2000


def _mesh():
    return plsc.VectorSubcoreMesh(core_axis_name="c", subcore_axis_name="s")


@functools.cache
def _agg_kernel(tab_rows, et, n_acc, f):
    nb = et // K
    R = 4
    QI = 8
    assert et % (QI * K) == 0 and n_acc % NS == 0
    rpt = n_acc // NS
    chunks = [(t * K, K) for t in range(rpt // K)]
    if rpt % K:
        chunks.append((rpt // K * K, rpt % K))

    @functools.partial(
        pl.kernel,
        out_type=jax.ShapeDtypeStruct((NC * n_acc, f), jnp.float32),
        mesh=_mesh(),
        scratch_types=[
            [pltpu.VMEM((K,), jnp.int32) for _ in range(QI)],
            [pltpu.VMEM((K,), jnp.int32) for _ in range(QI)],
            [pltpu.VMEM((K, f), jnp.float32) for _ in range(R)],
            pltpu.VMEM_SHARED((n_acc, f), jnp.float32),
            [pltpu.SemaphoreType.DMA for _ in range(QI)],
            [pltpu.SemaphoreType.DMA for _ in range(QI)],
            [pltpu.SemaphoreType.DMA for _ in range(R)],
            [pltpu.SemaphoreType.DMA for _ in range(R)],
        ],
    )
    def agg(table, src, dst, zrows, out,
            srcv, dstv, rows, acc, isem, jsem, gsem, ssem):
        c = lax.axis_index("c")
        s = lax.axis_index("s")

        pltpu.sync_copy(zrows, rows[0])
        for off, sz in chunks:
            pltpu.async_copy(
                rows[0].at[pl.ds(0, sz)],
                acc.at[pl.ds(s * rpt + off, sz)], gsem[0])
        for off, sz in chunks:
            pltpu.make_async_copy(
                rows[0].at[pl.ds(0, sz)],
                acc.at[pl.ds(s * rpt, sz)], gsem[0]).wait()
        plsc.subcore_barrier()

        ebase = (c * NS + s) * et

        def fire_idx(b, q):
            o = pl.multiple_of(ebase + b * K, K)
            pltpu.async_copy(src.at[pl.ds(o, K)], srcv[q], isem[q])
            pltpu.async_copy(dst.at[pl.ds(o, K)], dstv[q], jsem[q])

        def wait(sem, src_ref, dst_ref):
            pltpu.make_async_copy(src_ref, dst_ref, sem).wait()

        def fire_gather(b, q, j):
            o = pl.multiple_of(ebase + b * K, K)
            wait(isem[q], src.at[pl.ds(o, K)], srcv[q])
            pltpu.async_copy(table.at[srcv[q]], rows[j], gsem[j])

        for q in range(QI // 2):
            fire_idx(q, q)
        fire_gather(0, 0, 0)
        fire_gather(1, 1, 1)

        @pl.loop(0, nb, step=QI)
        def _edges(i):
            for u in range(QI):
                b = i + u
                j = u % R
                q = u
                j2 = (u + 2) % R
                q2 = (u + 2) % QI
                qr = (u + 4) % QI
                ob = pl.multiple_of(ebase + b * K, K)

                wait(gsem[j], table.at[srcv[q]], rows[j])
                wait(jsem[q], dst.at[pl.ds(ob, K)], dstv[q])
                pltpu.async_copy(rows[j], acc.at[dstv[q]], ssem[j], add=True)

                @pl.when(b + 2 < nb)
                def _next_gather():
                    @pl.when(b >= 2)
                    def _slot_free():
                        wait(ssem[j2], rows[j2], acc.at[dstv[q2]])
                    fire_gather(b + 2, q2, j2)

                @pl.when(b + 4 < nb)
                def _refill():
                    fire_idx(b + 4, qr)

        for j in range(R):
            wait(ssem[j], rows[j], acc.at[dstv[QI // 2 + j]])
        plsc.subcore_barrier()

        for t, (off, sz) in enumerate(chunks):
            j = t % 2
            if t >= 2:
                po, psz = chunks[t - 2]
                wait(gsem[j], rows[j].at[pl.ds(0, psz)],
                     out.at[pl.ds(c * n_acc, psz)])
            r0 = s * rpt + off
            pltpu.sync_copy(acc.at[pl.ds(r0, sz)], rows[j].at[pl.ds(0, sz)])
            pltpu.async_copy(rows[j].at[pl.ds(0, sz)],
                             out.at[pl.ds(c * n_acc + r0, sz)], gsem[j])
        for t in range(max(0, len(chunks) - 2), len(chunks)):
            off, sz = chunks[t]
            wait(gsem[t % 2], rows[t % 2].at[pl.ds(0, sz)],
                 out.at[pl.ds(c * n_acc, sz)])

    return agg


KD = 128


@functools.cache
def _deg_kernel(et, nh):
    nb = et // KD
    seg = nh // NS
    assert nh % (NS * L) == 0

    @functools.partial(
        pl.kernel,
        out_type=jax.ShapeDtypeStruct((NC * nh,), jnp.float32),
        mesh=_mesh(),
        compiler_params=pltpu.CompilerParams(needs_layout_passes=False),
        scratch_types=[
            pltpu.VMEM((nh,), jnp.float32),
            [pltpu.VMEM((KD,), jnp.int32) for _ in range(2)],
            pltpu.VMEM((seg,), jnp.float32),
            pltpu.VMEM((seg,), jnp.float32),
            pltpu.VMEM_SHARED((NS * nh,), jnp.float32),
            [pltpu.SemaphoreType.DMA for _ in range(2)],
        ],
    )
    def degk(dst, out, hist, dstv, tmp, accv, spm, dsem):
        c = lax.axis_index("c")
        s = lax.axis_index("s")

        @pl.loop(0, nh // L)
        def _zero(i):
            hist[pl.ds(pl.multiple_of(i * L, L), L)] = jnp.zeros((L,), jnp.float32)

        ebase = (c * NS + s) * et

        def fire(i, j):
            off = pl.multiple_of(ebase + i * KD, KD)
            pltpu.async_copy(dst.at[pl.ds(off, KD)], dstv[j], dsem[j])

        fire(0, 0)

        @pl.loop(0, nb, step=2)
        def _count(i):
            for u in range(2):
                b = i + u
                j = u
                o = pl.multiple_of(ebase + b * KD, KD)
                pltpu.make_async_copy(dst.at[pl.ds(o, KD)], dstv[j],
                                      dsem[j]).wait()

                @pl.when(b + 1 < nb)
                def _pf():
                    fire(b + 1, 1 - j)
                for k in range(KD // L):
                    idx = dstv[j][pl.ds(k * L, L)]
                    plsc.addupdate_scatter(hist, [idx],
                                           jnp.ones((L,), jnp.float32))

        pltpu.sync_copy(hist, spm.at[pl.ds(s * nh, nh)])
        plsc.subcore_barrier()

        pltpu.sync_copy(spm.at[pl.ds(s * seg, seg)], accv)
        for j in range(1, NS):
            pltpu.sync_copy(spm.at[pl.ds(j * nh + s * seg, seg)], tmp)

            @pl.loop(0, seg // L)
            def _red(m):
                o = pl.multiple_of(m * L, L)
                accv[pl.ds(o, L)] = accv[pl.ds(o, L)] + tmp[pl.ds(o, L)]

        pltpu.sync_copy(accv, out.at[pl.ds(c * nh + s * seg, seg)])

    return degk


def _tc_params():
    return pltpu.CompilerParams(vmem_limit_bytes=60 * 1024 * 1024)


def _dot(a, b):
    return lax.dot_general(a, b, (((1,), (0,)), ((), ())),
                           precision=lax.Precision.HIGHEST)


def kernel(x, edge_index, W1, b1, g1, bt1, W2, b2, g2, bt2, W3, b3):
    n, fin = x.shape
    e = edge_index.shape[1]
    h = W1.shape[1]
    fo = W3.shape[1]
    n_acc = -(-n // (NS * 8)) * (NS * 8)
    nh = -(-n // (NS * L)) * (NS * L)
    src = edge_index[0].astype(jnp.int32)
    dst = edge_index[1].astype(jnp.int32)
    zrows = jnp.zeros((K, fin), jnp.float32)
    nblk = n // BR
    assert n % BR == 0

    def chunked(vals, T, et, is_dst):
        k = jnp.arange(et, dtype=jnp.int32)[None, :]
        padv = n + (k % (n_acc - n)) if is_dst else k % K
        if e % T == 0:
            body = vals.reshape(T, e // T)
            padcols = jnp.broadcast_to(padv[:, e // T:], (T, et - e // T))
            return jnp.concatenate([body, padcols], axis=1).reshape(-1)
        base, rem = divmod(e, T)
        t = jnp.arange(T, dtype=jnp.int32)[:, None]
        cnt = base + (t < rem).astype(jnp.int32)
        ridx = t * base + jnp.minimum(t, rem) + k
        return jnp.where(k < cnt, vals[jnp.clip(ridx, 0, e - 1)],
                         padv).reshape(-1)

    et_deg = -(-e // (NC * NS * 2 * KD)) * (2 * KD)
    dst_deg = chunked(dst, NC * NS, et_deg, True)

    et_es = -(-e // (NC * NS * 8 * K)) * (8 * K)
    src_es = chunked(src, NC * NS, et_es, False)
    dst_es = chunked(dst, NC * NS, et_es, True)

    et_fs = -(-e // (NS * 8 * K)) * (8 * K)
    srcp = chunked(src, NS, et_fs, False)
    dstp = chunked(dst, NS, et_fs, True)
    src_fs = jnp.concatenate([srcp, srcp + n])
    dst_fs = jnp.concatenate([dstp, dstp])

    col = lambda w: pl.BlockSpec((1, w), lambda i: (0, 0))
    rows = lambda w: pl.BlockSpec((BR, w), lambda i: (i, 0))
    pair = pl.BlockSpec((NC, BR, fin), lambda i: (0, i, 0))

    degp = _deg_kernel(et_deg, nh)(dst_deg).reshape(NC, nh, 1)

    def t1(degp_ref, x_ref, dinv_ref, y0_ref):
        deg = degp_ref[0] + degp_ref[1] + 1.0
        dinv = jnp.where(deg > 0.0, lax.rsqrt(deg), 0.0)
        dinv_ref[...] = dinv
        y0_ref[...] = dinv * x_ref[...]

    dinv, y0 = pl.pallas_call(
        t1,
        grid=(nblk,),
        in_specs=[pl.BlockSpec((NC, BR, 1), lambda i: (0, i, 0)), rows(fin)],
        out_specs=[rows(1), rows(fin)],
        out_shape=[jax.ShapeDtypeStruct((n, 1), jnp.float32),
                   jax.ShapeDtypeStruct((n, fin), jnp.float32)],
        compiler_params=_tc_params(),
    )(degp, x)

    s0 = _agg_kernel(n, et_es, n_acc, fin)(y0, src_es, dst_es, zrows)
    s0 = s0.reshape(NC, n_acc, fin)

    def accum_stats(z, s_ref, q_ref):
        @pl.when(pl.program_id(0) == 0)
        def _init():
            s_ref[...] = jnp.zeros_like(s_ref)
            q_ref[...] = jnp.zeros_like(q_ref)
        s_ref[...] += jnp.sum(z, axis=0, keepdims=True)
        q_ref[...] += jnp.sum(z * z, axis=0, keepdims=True)

    def bn_relu(z, sm, sq, g, bt):
        m = sm * (1.0 / n)
        v = sq * (1.0 / n) - m * m
        return jnp.maximum((z - m) / jnp.sqrt(v + 1e-5) * g + bt, 0.0)

    def t2a(s0_ref, y0_ref, dinv_ref, w1_ref, b1_ref, z1_ref, sm_ref, sq_ref):
        agg0 = dinv_ref[...] * (s0_ref[0] + s0_ref[1] + y0_ref[...])
        z1 = _dot(agg0, w1_ref[...]) + b1_ref[...]
        z1_ref[...] = z1
        accum_stats(z1, sm_ref, sq_ref)

    z1, m1, q1 = pl.pallas_call(
        t2a,
        grid=(nblk,),
        in_specs=[pair, rows(fin), rows(1),
                  pl.BlockSpec((fin, h), lambda i: (0, 0)), col(h)],
        out_specs=[rows(h), col(h), col(h)],
        out_shape=[jax.ShapeDtypeStruct((n, h), jnp.float32),
                   jax.ShapeDtypeStruct((1, h), jnp.float32),
                   jax.ShapeDtypeStruct((1, h), jnp.float32)],
        compiler_params=_tc_params(),
    )(s0, y0, dinv, W1, b1.reshape(1, h))

    def t2b(z_ref, s_ref, q_ref, g_ref, bt_ref, dinv_ref, w_ref, y2_ref):
        h1 = bn_relu(z_ref[...], s_ref[...], q_ref[...], g_ref[...],
                     bt_ref[...])
        y2 = dinv_ref[...] * _dot(h1, w_ref[...])
        y2_ref[0] = y2[:, :fin]
        y2_ref[1] = y2[:, fin:]

    y2pair = pl.pallas_call(
        t2b,
        grid=(nblk,),
        in_specs=[rows(h), col(h), col(h), col(h), col(h), rows(1),
                  pl.BlockSpec((h, h), lambda i: (0, 0))],
        out_specs=pair,
        out_shape=jax.ShapeDtypeStruct((NC, n, fin), jnp.float32),
        compiler_params=_tc_params(),
    )(z1, m1, q1, g1.reshape(1, h), bt1.reshape(1, h), dinv, W2)

    s2 = _agg_kernel(NC * n, et_fs, n_acc, fin)(
        y2pair.reshape(NC * n, fin), src_fs, dst_fs, zrows)
    s2 = s2.reshape(NC, n_acc, fin)

    def t3a(s2_ref, y2_ref, dinv_ref, b2_ref, z2_ref, sm_ref, sq_ref):
        ssum = jnp.concatenate(
            [s2_ref[0] + y2_ref[0], s2_ref[1] + y2_ref[1]], axis=1)
        z2 = dinv_ref[...] * ssum + b2_ref[...]
        z2_ref[...] = z2
        accum_stats(z2, sm_ref, sq_ref)

    z2, m2, q2 = pl.pallas_call(
        t3a,
        grid=(nblk,),
        in_specs=[pair, pair, rows(1), col(h)],
        out_specs=[rows(h), col(h), col(h)],
        out_shape=[jax.ShapeDtypeStruct((n, h), jnp.float32),
                   jax.ShapeDtypeStruct((1, h), jnp.float32),
                   jax.ShapeDtypeStruct((1, h), jnp.float32)],
        compiler_params=_tc_params(),
    )(s2, y2pair, dinv, b2.reshape(1, h))

    def t3b(z_ref, s_ref, q_ref, g_ref, bt_ref, dinv_ref, w_ref, y3_ref):
        h2 = bn_relu(z_ref[...], s_ref[...], q_ref[...], g_ref[...],
                     bt_ref[...])
        y3_ref[...] = dinv_ref[...] * _dot(h2, w_ref[...])

    y3 = pl.pallas_call(
        t3b,
        grid=(nblk,),
        in_specs=[rows(h), col(h), col(h), col(h), col(h), rows(1),
                  pl.BlockSpec((h, fo), lambda i: (0, 0))],
        out_specs=rows(fo),
        out_shape=jax.ShapeDtypeStruct((n, fo), jnp.float32),
        compiler_params=_tc_params(),
    )(z2, m2, q2, g2.reshape(1, h), bt2.reshape(1, h), dinv, W3)

    s3 = _agg_kernel(n, et_es, n_acc, fo)(y3, src_es, dst_es, zrows)
    s3 = s3.reshape(NC, n_acc, fo)

    def t4(s3_ref, y3_ref, dinv_ref, b3_ref, o_ref):
        o_ref[...] = dinv_ref[...] * (s3_ref[0] + s3_ref[1] + y3_ref[...]) \
            + b3_ref[...]

    out = pl.pallas_call(
        t4,
        grid=(nblk,),
        in_specs=[pl.BlockSpec((NC, BR, fo), lambda i: (0, i, 0)), rows(fo),
                  rows(1), col(fo)],
        out_specs=rows(fo),
        out_shape=jax.ShapeDtypeStruct((n, fo), jnp.float32),
        compiler_params=_tc_params(),
    )(s3, y3, dinv, b3.reshape(1, fo))
    return out
